# baseline (speedup 1.0000x reference)
import functools

import jax
import jax.numpy as jnp
from jax import lax
from jax.experimental import pallas as pl
from jax.experimental.pallas import tpu as pltpu
from jax.experimental.pallas import tpu_sc as plsc

N_LAYER, B, H, S, D = 4, 8, 8, 2048, 128
ROWS = B * H * S
NW = 32
RPW = ROWS // NW
CH = 256
NCH = RPW // CH
NBUF = 3
BH_PER_W = (B * H) // NW


def _sc_body(src, val, params_h, dst, pbuf, bufs, rbuf, gsems, ssems):
    w = lax.axis_index("s") * 2 + lax.axis_index("c")
    pltpu.sync_copy(params_h, pbuf)
    pvec = pbuf[...]
    layer_base = pl.multiple_of(pvec[0], 8)
    pos = pvec[1]
    pos_div = pos // CH
    pos_mod = pos % CH
    base = w * RPW

    pltpu.sync_copy(val.at[pl.ds(w * BH_PER_W, BH_PER_W), :], rbuf)
    vrows = [[rbuf[j, pl.ds(16 * k, 16)] for k in range(D // 16)]
             for j in range(BH_PER_W)]

    def gather(c, slot):
        r = base + c * CH
        return pltpu.make_async_copy(
            src.at[pl.ds(layer_base + r, CH), :], bufs.at[slot], gsems.at[slot])

    def scatter(c, slot):
        r = base + c * CH
        return pltpu.make_async_copy(
            bufs.at[slot], dst.at[pl.ds(r, CH), :], ssems.at[slot])

    for p in range(NBUF - 1):
        gather(p, p).start()
    for i in range(NCH):
        slot = i % NBUF
        gather(i, slot).wait()
        for j in range(BH_PER_W):
            @pl.when(i == j * (S // CH) + pos_div)
            def _():
                for k in range(D // 16):
                    bufs[slot, pos_mod, pl.ds(16 * k, 16)] = vrows[j][k]
        scatter(i, slot).start()
        nxt = i + NBUF - 1
        if nxt < NCH:
            nslot = nxt % NBUF
            if nxt >= NBUF:
                scatter(nxt - NBUF, nslot).wait()
            gather(nxt, nslot).start()
    for i in range(max(0, NCH - NBUF), NCH):
        scatter(i, i % NBUF).wait()


def _sc_copy(src2, val2, params):
    f = pl.kernel(
        _sc_body,
        out_type=jax.ShapeDtypeStruct((ROWS, D), jnp.float32),
        mesh=plsc.VectorSubcoreMesh(core_axis_name="c", subcore_axis_name="s"),
        scratch_types=(
            pltpu.VMEM((16,), jnp.int32),
            pltpu.VMEM((NBUF, CH, D), jnp.float32),
            pltpu.VMEM((BH_PER_W, D), jnp.float32),
            pltpu.SemaphoreType.DMA((NBUF,)),
            pltpu.SemaphoreType.DMA((NBUF,)),
        ),
    )
    return f(src2, val2, params)


def _tc_body(pref, src_ref, val_ref, o_ref):
    o_ref[...] = src_ref[...]
    o_ref[pl.ds(pref[1], 1), :] = val_ref[0]


def _tc_copy(src2, val2, params_tc):
    grid_spec = pltpu.PrefetchScalarGridSpec(
        num_scalar_prefetch=1,
        grid=(B * H,),
        in_specs=[
            pl.BlockSpec((S, D), lambda g, p: (p[0] + g, 0)),
            pl.BlockSpec((1, 1, D), lambda g, p: (g, 0, 0)),
        ],
        out_specs=pl.BlockSpec((S, D), lambda g, p: (g, 0)),
    )
    return pl.pallas_call(
        _tc_body,
        grid_spec=grid_spec,
        out_shape=jax.ShapeDtypeStruct((ROWS, D), jnp.float32),
    )(params_tc, src2, val2.reshape(B * H, 1, D))


@jax.jit
def _update(kc2, vc2, kval2, vval2, params, params_tc):
    k2 = _sc_copy(kc2, kval2, params)
    v2 = _tc_copy(vc2, vval2, params_tc)
    return k2, v2


def kernel(k_cache, v_cache, layer_idx, input_pos, k_val, v_val):
    layer_idx = jnp.asarray(layer_idx, jnp.int32)
    input_pos = jnp.asarray(input_pos, jnp.int32)
    kc2 = k_cache.reshape(N_LAYER * ROWS, D)
    vc2 = v_cache.reshape(N_LAYER * ROWS, D)
    kval2 = k_val.reshape(B * H, D)
    vval2 = v_val.reshape(B * H, D)
    params = jnp.zeros((16,), jnp.int32)
    params = params.at[0].set(layer_idx * ROWS).at[1].set(input_pos)
    params_tc = jnp.stack([layer_idx * (B * H), input_pos])
    k2, v2 = _update(kc2, vc2, kval2, vval2, params, params_tc)
    return (k2.reshape(B, H, S, D), v2.reshape(B, H, S, D))
